# Initial kernel scaffold; baseline (speedup 1.0000x reference)
import jax
import jax.numpy as jnp
from jax.experimental import pallas as pl


def kernel(x, table):
    raise NotImplementedError("write your pallas kernel here")


# speedup vs baseline: 1.4647x; 1.4647x over previous
import functools

import jax
import jax.numpy as jnp
from jax import lax
from jax.experimental import pallas as pl
from jax.experimental.pallas import tpu as pltpu
from jax.experimental.pallas import tpu_sc as plsc

B = 4096
F = 26
D = 64
LANES = 16
NUM_WORKERS = 32
ROWS_PER_W = B // NUM_WORKERS
CHUNK_ROWS = 4
IDX_PER_CHUNK = CHUNK_ROWS * F
NCHUNK = ROWS_PER_W // CHUNK_ROWS
NBUF = 2


def _emb_body(idx_hbm, table_hbm, out_hbm, idx_v, rows_v, out_v, sem0, sem1):
    sems = (sem0, sem1)
    cid = lax.axis_index("c")
    sid = lax.axis_index("s")
    wid = sid * 2 + cid
    ibase = wid * (ROWS_PER_W * F)
    obase = wid * ROWS_PER_W

    def load_idx(c, buf):
        off = pl.multiple_of(ibase + c * IDX_PER_CHUNK, 8)
        pltpu.sync_copy(idx_hbm.at[pl.ds(off, IDX_PER_CHUNK)], idx_v.at[buf])

    def start_gather(buf):
        pltpu.make_async_copy(
            table_hbm.at[idx_v.at[buf]], rows_v.at[buf], sems[buf]
        ).start()

    def wait_gather(buf):
        pltpu.make_async_copy(
            table_hbm.at[idx_v.at[buf]], rows_v.at[buf], sems[buf]
        ).wait()

    def compute_store(c, buf):
        for i in range(CHUNK_ROWS):
            for d in range(D // LANES):
                sl = pl.ds(d * LANES, LANES)
                acc = rows_v[buf, i * F, sl]
                for f in range(1, F):
                    acc = acc + rows_v[buf, i * F + f, sl]
                out_v[i, sl] = acc
        orow = obase + c * CHUNK_ROWS
        pltpu.sync_copy(out_v, out_hbm.at[pl.ds(orow, CHUNK_ROWS)])

    load_idx(0, 0)
    start_gather(0)

    def outer(it, carry):
        c2 = it * NBUF
        for b in range(NBUF):
            c = c2 + b
            nxt = c + 1

            @pl.when(nxt < NCHUNK)
            def _():
                load_idx(nxt, 1 - b)
                start_gather(1 - b)

            wait_gather(b)
            compute_store(c, b)
        return carry

    lax.fori_loop(0, NCHUNK // NBUF, outer, 0)


def kernel(x, table):
    idx_flat = x.reshape(-1)
    mesh = plsc.VectorSubcoreMesh(core_axis_name="c", subcore_axis_name="s")
    k = functools.partial(
        pl.kernel,
        mesh=mesh,
        out_type=jax.ShapeDtypeStruct((B, D), jnp.float32),
        scratch_types=[
            pltpu.VMEM((NBUF, IDX_PER_CHUNK), jnp.int32),
            pltpu.VMEM((NBUF, IDX_PER_CHUNK, D), jnp.float32),
            pltpu.VMEM((CHUNK_ROWS, D), jnp.float32),
            pltpu.SemaphoreType.DMA,
            pltpu.SemaphoreType.DMA,
        ],
        compiler_params=pltpu.CompilerParams(use_tc_tiling_on_sc=False),
    )(_emb_body)
    return k(idx_flat, table)


# speedup vs baseline: 1.5535x; 1.0606x over previous
import functools

import jax
import jax.numpy as jnp
from jax import lax
from jax.experimental import pallas as pl
from jax.experimental.pallas import tpu as pltpu
from jax.experimental.pallas import tpu_sc as plsc

B = 4096
F = 26
D = 64
LANES = 16
NUM_WORKERS = 32
ROWS_PER_W = B // NUM_WORKERS
CHUNK_ROWS = 16
NSUB = 4
SUB_IDX = CHUNK_ROWS * F // NSUB
IDX_PER_CHUNK = CHUNK_ROWS * F
NCHUNK = ROWS_PER_W // CHUNK_ROWS
NBUF = 2


def _emb_body(idx_hbm, table_hbm, out_hbm, idx_v, rows_v, out_v, sem0, sem1):
    sems = (sem0, sem1)
    cid = lax.axis_index("c")
    sid = lax.axis_index("s")
    wid = sid * 2 + cid
    ibase = wid * (ROWS_PER_W * F)
    obase = wid * ROWS_PER_W

    def load_idx(c, buf):
        off = pl.multiple_of(ibase + c * IDX_PER_CHUNK, 8)
        pltpu.sync_copy(
            idx_hbm.at[pl.ds(off, IDX_PER_CHUNK)], idx_v.at[buf]
        )

    def start_gathers(buf):
        for j in range(NSUB):
            pltpu.make_async_copy(
                table_hbm.at[idx_v.at[buf, pl.ds(j * SUB_IDX, SUB_IDX)]],
                rows_v.at[buf, pl.ds(j * SUB_IDX, SUB_IDX)],
                sems[buf],
            ).start()

    def wait_gathers(buf):
        for j in range(NSUB):
            pltpu.make_async_copy(
                table_hbm.at[idx_v.at[buf, pl.ds(j * SUB_IDX, SUB_IDX)]],
                rows_v.at[buf, pl.ds(j * SUB_IDX, SUB_IDX)],
                sems[buf],
            ).wait()

    def compute_store(c, buf):
        def row_body(i, carry):
            r0 = i * F
            for d in range(D // LANES):
                sl = pl.ds(d * LANES, LANES)
                acc = None
                for f in range(F):
                    v = rows_v[buf, r0 + f, sl]
                    acc = v if acc is None else acc + v
                out_v[i, sl] = acc
            return carry

        lax.fori_loop(0, CHUNK_ROWS, row_body, 0)
        orow = obase + c * CHUNK_ROWS
        pltpu.sync_copy(out_v, out_hbm.at[pl.ds(orow, CHUNK_ROWS)])

    load_idx(0, 0)
    start_gathers(0)

    def outer(it, carry):
        c2 = it * NBUF
        for b in range(NBUF):
            c = c2 + b
            nxt = c + 1

            @pl.when(nxt < NCHUNK)
            def _():
                load_idx(nxt, 1 - b)
                start_gathers(1 - b)

            wait_gathers(b)
            compute_store(c, b)
        return carry

    lax.fori_loop(0, NCHUNK // NBUF, outer, 0)


def kernel(x, table):
    idx_flat = x.reshape(-1)
    mesh = plsc.VectorSubcoreMesh(core_axis_name="c", subcore_axis_name="s")
    k = functools.partial(
        pl.kernel,
        mesh=mesh,
        out_type=jax.ShapeDtypeStruct((B, D), jnp.float32),
        scratch_types=[
            pltpu.VMEM((NBUF, IDX_PER_CHUNK), jnp.int32),
            pltpu.VMEM((NBUF, IDX_PER_CHUNK, D), jnp.float32),
            pltpu.VMEM((CHUNK_ROWS, D), jnp.float32),
            pltpu.SemaphoreType.DMA,
            pltpu.SemaphoreType.DMA,
        ],
        compiler_params=pltpu.CompilerParams(use_tc_tiling_on_sc=False),
    )(_emb_body)
    return k(idx_flat, table)
